# baseline (speedup 1.0000x reference)
import functools

import jax
import jax.numpy as jnp
from jax import lax
from jax.experimental import pallas as pl
from jax.experimental.pallas import tpu as pltpu
from jax.experimental.pallas import tpu_sc as plsc

LANES = 16


def _dechunk_sc(zf, pf, bf, B, T, L, D):
    info = plsc.get_sparse_core_info()
    NC, NS = info.num_cores, info.num_subcores
    NW = NC * NS
    CPB = NW // B
    TCH = T // CPB
    S = 32
    NSUB = TCH // S
    NVC = TCH // LANES
    UD = D // LANES

    mesh = plsc.VectorSubcoreMesh(core_axis_name="c", subcore_axis_name="s")

    @functools.partial(
        pl.kernel,
        out_type=jax.ShapeDtypeStruct((B * T, D), jnp.float32),
        mesh=mesh,
        scratch_types=[
            pltpu.VMEM((T + LANES,), jnp.int32),
            pltpu.VMEM((TCH + LANES,), jnp.float32),
            pltpu.VMEM((TCH + 2 * LANES,), jnp.int32),
            pltpu.VMEM((T + 2 * LANES,), jnp.int32),
            pltpu.VMEM((LANES,), jnp.int32),
            pltpu.VMEM((16, LANES), jnp.int32),
            pltpu.VMEM_SHARED((16, LANES), jnp.int32),
            pltpu.VMEM((8, D), jnp.float32),
            pltpu.VMEM((S, D), jnp.float32),
            pltpu.VMEM((S, D), jnp.float32),
            pltpu.VMEM((S, D), jnp.float32),
            pltpu.VMEM((S, D), jnp.float32),
            pltpu.SemaphoreType.DMA,
            pltpu.SemaphoreType.DMA,
            pltpu.SemaphoreType.DMA,
            pltpu.SemaphoreType.DMA,
            pltpu.SemaphoreType.DMA,
        ],
    )
    def body(z_hbm, p_hbm, b_hbm, out_hbm, b_v, p_v, c_v, idxf_v, idxp_v,
             tot_v, tot_sh, prev0_v, rowsA, rowsB, outbA, outbB,
             gsemA, gsemB, osemA, osemB, psem):
        sid = lax.axis_index("s")
        wid = sid * NC + lax.axis_index("c")
        batch = wid % B
        chunk = wid // B
        start = chunk * TCH
        obase = batch * T + start

        pltpu.sync_copy(b_hbm.at[pl.ds(batch * T, T)],
                        b_v.at[pl.ds(LANES, T)])
        pltpu.sync_copy(p_hbm.at[pl.ds(batch * T + start, TCH)],
                        p_v.at[pl.ds(0, TCH)])

        zbase = batch * L
        zeros = jnp.zeros((LANES,), jnp.int32)
        b_v[pl.ds(0, LANES)] = zeros

        @pl.when(chunk == 0)
        def _():
            lane = jnp.arange(LANES, dtype=jnp.int32)
            p0 = p_v[pl.ds(0, LANES)]
            p_v[pl.ds(0, LANES)] = jnp.where(lane == 0, 1.0, p0)

        c_v[pl.ds(0, LANES)] = zeros

        def shift_body(m, carry):
            c_v[pl.ds(LANES + m * LANES, LANES)] = b_v[pl.ds(
                LANES - 1 + start + m * LANES, LANES)]
            return carry

        lax.fori_loop(0, NVC, shift_body, jnp.int32(0))

        s = 1
        while s < TCH:
            lo = s // LANES

            def scan_body(jj, carry, s=s, lo=lo):
                j = NVC - 1 - jj
                o = LANES + j * LANES
                c_v[pl.ds(o, LANES)] = (c_v[pl.ds(o, LANES)] +
                                        c_v[pl.ds(o - s, LANES)])
                return carry

            lax.fori_loop(0, NVC - lo, scan_body, jnp.int32(0))
            s *= 2

        pltpu.sync_copy(c_v.at[pl.ds(LANES + TCH - LANES, LANES)],
                        tot_sh.at[sid])
        plsc.subcore_barrier()
        pltpu.sync_copy(tot_sh, tot_v)
        off = jnp.int32(0)
        for k in range(1, CPB):
            vk = tot_v[jnp.maximum(sid - 4 * k, 0), pl.ds(0, LANES)][15]
            off = off + jnp.where(chunk >= k, vk, 0)

        idxp_v[pl.ds(0, LANES)] = (
            jnp.minimum(c_v[pl.ds(LANES - 1, LANES)] + off, L - 1) + zbase)

        def idx_body(m, carry):
            o = m * LANES
            e = c_v[pl.ds(LANES + o, LANES)] + off
            idxf_v[pl.ds(LANES + start + o, LANES)] = (
                jnp.minimum(e, L - 1) + zbase)
            return carry

        lax.fori_loop(0, NVC, idx_body, jnp.int32(0))

        pltpu.async_copy(z_hbm.at[idxp_v.at[pl.ds(0, 8)]], prev0_v,
                         psem).wait()

        def g_src(i):
            return z_hbm.at[idxf_v.at[pl.ds(LANES + start + i * S, S)]]

        pltpu.async_copy(g_src(0), rowsA, gsemA)
        pltpu.async_copy(g_src(1), rowsB, gsemB)

        def blend(s0, rows, outb, prev):
            def rb(k, prev, rows=rows, outb=outb):
                pk = p_v[pl.ds(s0 + k, LANES)][0]
                qk = 1.0 - pk
                new = []
                for u in range(UD):
                    sl = pl.ds(u * LANES, LANES)
                    cu = rows[k, sl]
                    outb[k, sl] = pk * cu + qk * prev[u]
                    new.append(cu)
                return tuple(new)

            return lax.fori_loop(0, S, rb, prev, unroll=2)

        def half(ii, i, rows, outb, gsem, osem, prev):
            s0 = i * S
            pltpu.make_async_copy(z_hbm.at[pl.ds(0, S)], rows, gsem).wait()

            @pl.when(ii > 0)
            def _():
                pltpu.make_async_copy(outb, out_hbm.at[pl.ds(0, S)],
                                      osem).wait()

            prev = blend(s0, rows, outb, prev)
            pltpu.async_copy(outb, out_hbm.at[pl.ds(obase + s0, S)], osem)

            @pl.when(i + 2 < NSUB)
            def _():
                pltpu.async_copy(g_src(i + 2), rows, gsem)

            return prev

        prev0 = tuple(prev0_v[0, pl.ds(u * LANES, LANES)] for u in range(UD))

        def pair_body(ii, prev):
            prev = half(ii, 2 * ii, rowsA, outbA, gsemA, osemA, prev)
            prev = half(ii, 2 * ii + 1, rowsB, outbB, gsemB, osemB, prev)
            return prev

        lax.fori_loop(0, NSUB // 2, pair_body, prev0)

        pltpu.make_async_copy(outbA, out_hbm.at[pl.ds(0, S)], osemA).wait()
        pltpu.make_async_copy(outbB, out_hbm.at[pl.ds(0, S)], osemB).wait()

    return body(zf, pf, bf)


def kernel(z, p, b, original_len):
    B, L, D = z.shape
    T = b.shape[1]
    zf = z.reshape(B * L, D)
    pf = p.reshape(B * T)
    bf = b.reshape(B * T)
    out = _dechunk_sc(zf, pf, bf, B, T, L, D)
    return out.reshape(B, T, D)


# speedup vs baseline: 1.1684x; 1.1684x over previous
import functools

import jax
import jax.numpy as jnp
from jax import lax
from jax.experimental import pallas as pl
from jax.experimental.pallas import tpu as pltpu
from jax.experimental.pallas import tpu_sc as plsc

LANES = 16


def _dechunk_sc(zf, pf, bf, B, T, L, D):
    info = plsc.get_sparse_core_info()
    NC, NS = info.num_cores, info.num_subcores
    NW = NC * NS
    CPB = NW // B
    TCH = T // CPB
    S = 32
    NSUB = TCH // S
    NVC = TCH // LANES
    UD = D // LANES

    mesh = plsc.VectorSubcoreMesh(core_axis_name="c", subcore_axis_name="s")

    @functools.partial(
        pl.kernel,
        out_type=jax.ShapeDtypeStruct((B * T, D), jnp.float32),
        mesh=mesh,
        scratch_types=[
            pltpu.VMEM((T + LANES,), jnp.int32),
            pltpu.VMEM((TCH + LANES,), jnp.float32),
            pltpu.VMEM((TCH + 2 * LANES,), jnp.int32),
            pltpu.VMEM((T + 2 * LANES,), jnp.int32),
            pltpu.VMEM((LANES,), jnp.int32),
            pltpu.VMEM((16, LANES), jnp.int32),
            pltpu.VMEM_SHARED((16, LANES), jnp.int32),
            pltpu.VMEM((8, D), jnp.float32),
            pltpu.VMEM((S, D), jnp.float32),
            pltpu.VMEM((S, D), jnp.float32),
            pltpu.VMEM((S, D), jnp.float32),
            pltpu.VMEM((S, D), jnp.float32),
            pltpu.SemaphoreType.DMA,
            pltpu.SemaphoreType.DMA,
            pltpu.SemaphoreType.DMA,
            pltpu.SemaphoreType.DMA,
            pltpu.SemaphoreType.DMA,
        ],
    )
    def body(z_hbm, p_hbm, b_hbm, out_hbm, b_v, p_v, c_v, idxf_v, idxp_v,
             tot_v, tot_sh, prev0_v, rowsA, rowsB, outbA, outbB,
             gsemA, gsemB, osemA, osemB, psem):
        sid = lax.axis_index("s")
        wid = sid * NC + lax.axis_index("c")
        batch = wid % B
        chunk = wid // B
        start = chunk * TCH
        obase = batch * T + start

        pltpu.sync_copy(b_hbm.at[pl.ds(batch * T, T)],
                        b_v.at[pl.ds(LANES, T)])
        pltpu.sync_copy(p_hbm.at[pl.ds(batch * T + start, TCH)],
                        p_v.at[pl.ds(0, TCH)])

        zbase = batch * L
        zeros = jnp.zeros((LANES,), jnp.int32)
        b_v[pl.ds(0, LANES)] = zeros

        @pl.when(chunk == 0)
        def _():
            lane = jnp.arange(LANES, dtype=jnp.int32)
            p0 = p_v[pl.ds(0, LANES)]
            p_v[pl.ds(0, LANES)] = jnp.where(lane == 0, 1.0, p0)

        c_v[pl.ds(0, LANES)] = zeros

        def shift_body(m, carry):
            c_v[pl.ds(LANES + m * LANES, LANES)] = b_v[pl.ds(
                LANES - 1 + start + m * LANES, LANES)]
            return carry

        lax.fori_loop(0, NVC, shift_body, jnp.int32(0))

        s = 1
        while s < TCH:
            lo = s // LANES

            def scan_body(jj, carry, s=s, lo=lo):
                j = NVC - 1 - jj
                o = LANES + j * LANES
                c_v[pl.ds(o, LANES)] = (c_v[pl.ds(o, LANES)] +
                                        c_v[pl.ds(o - s, LANES)])
                return carry

            lax.fori_loop(0, NVC - lo, scan_body, jnp.int32(0))
            s *= 2

        pltpu.sync_copy(c_v.at[pl.ds(LANES + TCH - LANES, LANES)],
                        tot_sh.at[sid])
        plsc.subcore_barrier()
        pltpu.sync_copy(tot_sh, tot_v)
        off = jnp.int32(0)
        for k in range(1, CPB):
            vk = tot_v[jnp.maximum(sid - 4 * k, 0), pl.ds(0, LANES)][15]
            off = off + jnp.where(chunk >= k, vk, 0)

        idxp_v[pl.ds(0, LANES)] = (
            jnp.minimum(c_v[pl.ds(LANES - 1, LANES)] + off, L - 1) + zbase)

        def idx_body(m, carry):
            o = m * LANES
            e = c_v[pl.ds(LANES + o, LANES)] + off
            idxf_v[pl.ds(LANES + start + o, LANES)] = (
                jnp.minimum(e, L - 1) + zbase)
            return carry

        lax.fori_loop(0, NVC, idx_body, jnp.int32(0))

        pltpu.async_copy(z_hbm.at[idxp_v.at[pl.ds(0, 8)]], prev0_v,
                         psem).wait()

        def g_src(i):
            return z_hbm.at[idxf_v.at[pl.ds(LANES + start + i * S, S)]]

        pltpu.async_copy(g_src(0), rowsA, gsemA)
        pltpu.async_copy(g_src(1), rowsB, gsemB)

        def blend(s0, rows, outb, prev):
            def rb(k, prev, rows=rows, outb=outb):
                pk = p_v[pl.ds(s0 + k, LANES)][0]
                qk = 1.0 - pk
                new = []
                for u in range(UD):
                    sl = pl.ds(u * LANES, LANES)
                    cu = rows[k, sl]
                    outb[k, sl] = pk * cu + qk * prev[u]
                    new.append(cu)
                return tuple(new)

            return lax.fori_loop(0, S, rb, prev)

        def half(ii, i, rows, outb, gsem, osem, prev):
            s0 = i * S
            pltpu.make_async_copy(z_hbm.at[pl.ds(0, S)], rows, gsem).wait()

            @pl.when(ii > 0)
            def _():
                pltpu.make_async_copy(outb, out_hbm.at[pl.ds(0, S)],
                                      osem).wait()

            prev = blend(s0, rows, outb, prev)
            pltpu.async_copy(outb, out_hbm.at[pl.ds(obase + s0, S)], osem)

            @pl.when(i + 2 < NSUB)
            def _():
                pltpu.async_copy(g_src(i + 2), rows, gsem)

            return prev

        prev0 = tuple(prev0_v[0, pl.ds(u * LANES, LANES)] for u in range(UD))

        def pair_body(ii, prev):
            prev = half(ii, 2 * ii, rowsA, outbA, gsemA, osemA, prev)
            prev = half(ii, 2 * ii + 1, rowsB, outbB, gsemB, osemB, prev)
            return prev

        lax.fori_loop(0, NSUB // 2, pair_body, prev0)

        pltpu.make_async_copy(outbA, out_hbm.at[pl.ds(0, S)], osemA).wait()
        pltpu.make_async_copy(outbB, out_hbm.at[pl.ds(0, S)], osemB).wait()

    return body(zf, pf, bf)


def kernel(z, p, b, original_len):
    B, L, D = z.shape
    T = b.shape[1]
    zf = z.reshape(B * L, D)
    pf = p.reshape(B * T)
    bf = b.reshape(B * T)
    out = _dechunk_sc(zf, pf, bf, B, T, L, D)
    return out.reshape(B, T, D)
